# baseline (speedup 1.0000x reference)
import functools

import jax
import jax.numpy as jnp
from jax import lax
from jax.experimental import pallas as pl
from jax.experimental.pallas import tpu as pltpu
from jax.experimental.pallas import tpu_sc as plsc

B = 1024
CTX = 50
DIM = 16
VOCAB = 100000

NC = 2
NS = 16
NW = NC * NS
ROWS_W = B // NW
IDX_W = ROWS_W * CTX
CHUNK = 100
NCHUNK = IDX_W // CHUNK


def _sc_gather_sum(inp_flat, emb_table):
    mesh = plsc.VectorSubcoreMesh(
        core_axis_name="c", subcore_axis_name="s", num_cores=NC, num_subcores=NS
    )

    @functools.partial(
        pl.kernel,
        out_type=jax.ShapeDtypeStruct((B, DIM), jnp.float32),
        mesh=mesh,
        scratch_types=[
            pltpu.VMEM((NCHUNK, CHUNK), jnp.int32),
            pltpu.VMEM((IDX_W, DIM), jnp.float32),
            pltpu.VMEM((ROWS_W, DIM), jnp.float32),
            pltpu.SemaphoreType.DMA,
        ],
        compiler_params=pltpu.CompilerParams(use_tc_tiling_on_sc=False),
    )
    def sc_kernel(inp_hbm, table_hbm, out_hbm, idx_v, rows_v, x_v, sem):
        wid = lax.axis_index("s") * NC + lax.axis_index("c")
        pltpu.sync_copy(inp_hbm.at[wid], idx_v)
        copies = [
            pltpu.async_copy(
                table_hbm.at[idx_v.at[j]],
                rows_v.at[pl.ds(j * CHUNK, CHUNK)],
                sem,
            )
            for j in range(NCHUNK)
        ]
        for c in copies:
            c.wait()
        for r in range(ROWS_W):
            def ctx_body(c, acc):
                return acc + rows_v[r * CTX + c, :]
            acc = lax.fori_loop(0, CTX, ctx_body, jnp.zeros((DIM,), jnp.float32),
                                unroll=10)
            x_v[r, :] = acc
        pltpu.sync_copy(x_v, out_hbm.at[pl.ds(wid * ROWS_W, ROWS_W)])

    return sc_kernel(inp_flat, emb_table)


VBLK = 1024
NVBLK = pl.cdiv(VOCAB, VBLK)


def _tc_matmul_t(xt, wt, b2):
    def mm(xt_ref, w_ref, b_ref, o_ref):
        acc = lax.dot_general(
            w_ref[...].astype(jnp.bfloat16), xt_ref[...].astype(jnp.bfloat16),
            (((0,), (0,)), ((), ())),
            preferred_element_type=jnp.float32,
        )
        o_ref[...] = acc + jnp.transpose(b_ref[...])

    return pl.pallas_call(
        mm,
        grid=(NVBLK,),
        in_specs=[
            pl.BlockSpec((DIM, B), lambda i: (0, 0)),
            pl.BlockSpec((DIM, VBLK), lambda i: (0, i)),
            pl.BlockSpec((1, VBLK), lambda i: (0, i)),
        ],
        out_specs=pl.BlockSpec((VBLK, B), lambda i: (i, 0)),
        out_shape=jax.ShapeDtypeStruct((VOCAB, B), jnp.float32),
    )(xt, wt, b2)


def kernel(inp, emb_table, W, b):
    inp_flat = inp.reshape(NW, NCHUNK, CHUNK).astype(jnp.int32)
    x = _sc_gather_sum(inp_flat, emb_table)
    yt = _tc_matmul_t(x.T, W.T, b.reshape(1, VOCAB))
    return yt.T


# speedup vs baseline: 1.1072x; 1.1072x over previous
import functools

import jax
import jax.numpy as jnp
from jax import lax
from jax.experimental import pallas as pl
from jax.experimental.pallas import tpu as pltpu
from jax.experimental.pallas import tpu_sc as plsc

B = 1024
CTX = 50
DIM = 16
VOCAB = 100000

NC = 2
NS = 16
NW = NC * NS
ROWS_W = B // NW
IDX_W = ROWS_W * CTX
CHUNK = 100
NCHUNK = IDX_W // CHUNK


def _sc_gather_sum(inp_flat, emb_table):
    mesh = plsc.VectorSubcoreMesh(
        core_axis_name="c", subcore_axis_name="s", num_cores=NC, num_subcores=NS
    )

    @functools.partial(
        pl.kernel,
        out_type=jax.ShapeDtypeStruct((B, DIM), jnp.float32),
        mesh=mesh,
        scratch_types=[
            pltpu.VMEM((NCHUNK, CHUNK), jnp.int32),
            pltpu.VMEM((IDX_W, DIM), jnp.float32),
            pltpu.VMEM((ROWS_W, DIM), jnp.float32),
            pltpu.SemaphoreType.DMA,
        ],
        compiler_params=pltpu.CompilerParams(use_tc_tiling_on_sc=False),
    )
    def sc_kernel(inp_hbm, table_hbm, out_hbm, idx_v, rows_v, x_v, sem):
        wid = lax.axis_index("s") * NC + lax.axis_index("c")
        pltpu.sync_copy(inp_hbm.at[wid], idx_v)
        copies = [
            pltpu.async_copy(
                table_hbm.at[idx_v.at[j]],
                rows_v.at[pl.ds(j * CHUNK, CHUNK)],
                sem,
            )
            for j in range(NCHUNK)
        ]
        for c in copies:
            c.wait()
        for r in range(ROWS_W):
            def ctx_body(c, acc):
                return acc + rows_v[r * CTX + c, :]
            acc = lax.fori_loop(0, CTX, ctx_body, jnp.zeros((DIM,), jnp.float32),
                                unroll=10)
            x_v[r, :] = acc
        pltpu.sync_copy(x_v, out_hbm.at[pl.ds(wid * ROWS_W, ROWS_W)])

    return sc_kernel(inp_flat, emb_table)


VBLK = 2048
NVBLK = pl.cdiv(VOCAB, VBLK)


def _tc_matmul_t(xt, wt, b2):
    def mm(xt_ref, w_ref, b_ref, o_ref):
        acc = lax.dot_general(
            w_ref[...].astype(jnp.bfloat16), xt_ref[...].astype(jnp.bfloat16),
            (((0,), (0,)), ((), ())),
            preferred_element_type=jnp.float32,
        )
        o_ref[...] = acc + jnp.transpose(b_ref[...])

    return pl.pallas_call(
        mm,
        grid=(NVBLK,),
        in_specs=[
            pl.BlockSpec((DIM, B), lambda i: (0, 0)),
            pl.BlockSpec((DIM, VBLK), lambda i: (0, i)),
            pl.BlockSpec((1, VBLK), lambda i: (0, i)),
        ],
        out_specs=pl.BlockSpec((VBLK, B), lambda i: (i, 0)),
        out_shape=jax.ShapeDtypeStruct((VOCAB, B), jnp.float32),
    )(xt, wt, b2)


def kernel(inp, emb_table, W, b):
    inp_flat = inp.reshape(NW, NCHUNK, CHUNK).astype(jnp.int32)
    x = _sc_gather_sum(inp_flat, emb_table)
    yt = _tc_matmul_t(x.T, W.T, b.reshape(1, VOCAB))
    return yt.T


# speedup vs baseline: 1.1104x; 1.0029x over previous
import functools

import jax
import jax.numpy as jnp
from jax import lax
from jax.experimental import pallas as pl
from jax.experimental.pallas import tpu as pltpu
from jax.experimental.pallas import tpu_sc as plsc

B = 1024
CTX = 50
DIM = 16
VOCAB = 100000

NC = 2
NS = 16
NW = NC * NS
ROWS_W = B // NW
IDX_W = ROWS_W * CTX
CHUNK = 100
NCHUNK = IDX_W // CHUNK


def _sc_gather_sum(inp_flat, emb_table):
    mesh = plsc.VectorSubcoreMesh(
        core_axis_name="c", subcore_axis_name="s", num_cores=NC, num_subcores=NS
    )

    @functools.partial(
        pl.kernel,
        out_type=jax.ShapeDtypeStruct((B, DIM), jnp.float32),
        mesh=mesh,
        scratch_types=[
            pltpu.VMEM((NCHUNK, CHUNK), jnp.int32),
            pltpu.VMEM((IDX_W, DIM), jnp.float32),
            pltpu.VMEM((ROWS_W, DIM), jnp.float32),
            pltpu.SemaphoreType.DMA,
        ],
        compiler_params=pltpu.CompilerParams(use_tc_tiling_on_sc=False),
    )
    def sc_kernel(inp_hbm, table_hbm, out_hbm, idx_v, rows_v, x_v, sem):
        wid = lax.axis_index("s") * NC + lax.axis_index("c")
        pltpu.sync_copy(inp_hbm.at[wid], idx_v)
        copies = [
            pltpu.async_copy(
                table_hbm.at[idx_v.at[j]],
                rows_v.at[pl.ds(j * CHUNK, CHUNK)],
                sem,
            )
            for j in range(NCHUNK)
        ]
        for c in copies:
            c.wait()
        for r in range(ROWS_W):
            def ctx_body(c, acc):
                return acc + rows_v[r * CTX + c, :]
            acc = lax.fori_loop(0, CTX, ctx_body, jnp.zeros((DIM,), jnp.float32),
                                unroll=10)
            x_v[r, :] = acc
        pltpu.sync_copy(x_v, out_hbm.at[pl.ds(wid * ROWS_W, ROWS_W)])

    return sc_kernel(inp_flat, emb_table)


VBLK = 2048
NVBLK = pl.cdiv(VOCAB, VBLK)


def _tc_matmul_t(xt, wt, b2):
    def mm(xt_ref, w_ref, b_ref, o_ref):
        acc = lax.dot_general(
            w_ref[...].astype(jnp.bfloat16), xt_ref[...].astype(jnp.bfloat16),
            (((0,), (0,)), ((), ())),
            preferred_element_type=jnp.float32,
        )
        o_ref[...] = acc + jnp.transpose(b_ref[...])

    return pl.pallas_call(
        mm,
        grid=(NVBLK,),
        in_specs=[
            pl.BlockSpec((DIM, B), lambda i: (0, 0)),
            pl.BlockSpec((DIM, VBLK), lambda i: (0, i)),
            pl.BlockSpec((1, VBLK), lambda i: (0, i)),
        ],
        out_specs=pl.BlockSpec((VBLK, B), lambda i: (i, 0)),
        out_shape=jax.ShapeDtypeStruct((VOCAB, B), jnp.float32),
        compiler_params=pltpu.CompilerParams(
            vmem_limit_bytes=128 * 1024 * 1024,
            dimension_semantics=("arbitrary",),
        ),
    )(xt, wt, b2)


def kernel(inp, emb_table, W, b):
    inp_flat = inp.reshape(NW, NCHUNK, CHUNK).astype(jnp.int32)
    x = _sc_gather_sum(inp_flat, emb_table)
    yt = _tc_matmul_t(x.T, W.T, b.reshape(1, VOCAB))
    return yt.T
